# baseline (device time: 1523720 ns/iter reference)
import jax
import jax.numpy as jnp
from jax import lax
from jax.experimental import pallas as pl
from jax.experimental.pallas import tpu as pltpu

N_DEV = 32


def kernel(x, w_mat):
    m_glob, k_shard = x.shape
    k2, n = w_mat.shape
    assert k2 == k_shard
    m_chunk = m_glob // N_DEV
    n_half = n // 2

    def body(x_ref, w_ref, out_ref,
             comm_r, comm_l,
             send_sems_r, recv_sems_r,
             send_sems_l, recv_sems_l,
             credit_r, credit_l):
        my = lax.axis_index("i")
        right = lax.rem(my + 1, N_DEV)
        left = lax.rem(my + N_DEV - 1, N_DEV)

        barrier_sem = pltpu.get_barrier_semaphore()
        for nbr in (left, right):
            pl.semaphore_signal(barrier_sem, inc=1, device_id=(nbr,),
                                device_id_type=pl.DeviceIdType.MESH)
        pl.semaphore_wait(barrier_sem, 2)

        def partial(c, lo):
            rows = x_ref[pl.ds(c * m_chunk, m_chunk), :]
            return jnp.dot(rows, w_ref[:, lo:lo + n_half],
                           preferred_element_type=jnp.float32)

        dirs = (
            (comm_r, send_sems_r, recv_sems_r, credit_r, right, left,
             lambda t: lax.rem(my + N_DEV - 1 - t, N_DEV), 0),
            (comm_l, send_sems_l, recv_sems_l, credit_l, left, right,
             lambda t: lax.rem(my + 1 + t, N_DEV), n_half),
        )

        for t in range(N_DEV - 1):
            s = t % 2
            r = (t + 1) % 2
            rdmas = []
            for comm, ssem, rsem, credit, dst, cdst, chunk_at, lo in dirs:
                p = partial(chunk_at(t), lo)
                if t == 0:
                    comm[s] = p
                else:
                    comm[s] = comm[s] + p
                if t >= 1:
                    pl.semaphore_wait(credit, 1)
                rdma = pltpu.make_async_remote_copy(
                    src_ref=comm.at[s],
                    dst_ref=comm.at[r],
                    send_sem=ssem.at[s],
                    recv_sem=rsem.at[r],
                    device_id=(dst,),
                    device_id_type=pl.DeviceIdType.MESH,
                )
                rdma.start()
                rdmas.append((rdma, credit, cdst))
            for rdma, credit, cdst in rdmas:
                rdma.wait()
                if t < N_DEV - 2:
                    pl.semaphore_signal(credit, inc=1, device_id=(cdst,),
                                        device_id_type=pl.DeviceIdType.MESH)

        fs = (N_DEV - 1) % 2
        out_ref[:, 0:n_half] = comm_r[fs] + partial(my, 0)
        out_ref[:, n_half:n] = comm_l[fs] + partial(my, n_half)

    return pl.pallas_call(
        body,
        out_shape=jax.ShapeDtypeStruct((m_chunk, n), jnp.float32),
        in_specs=[
            pl.BlockSpec(memory_space=pltpu.VMEM),
            pl.BlockSpec(memory_space=pltpu.VMEM),
        ],
        out_specs=pl.BlockSpec(memory_space=pltpu.VMEM),
        scratch_shapes=[
            pltpu.VMEM((2, m_chunk, n_half), jnp.float32),
            pltpu.VMEM((2, m_chunk, n_half), jnp.float32),
            pltpu.SemaphoreType.DMA((2,)),
            pltpu.SemaphoreType.DMA((2,)),
            pltpu.SemaphoreType.DMA((2,)),
            pltpu.SemaphoreType.DMA((2,)),
            pltpu.SemaphoreType.REGULAR,
            pltpu.SemaphoreType.REGULAR,
        ],
        compiler_params=pltpu.CompilerParams(collective_id=0),
    )(x, w_mat)


# device time: 1474152 ns/iter; 1.0336x vs baseline; 1.0336x over previous
import jax
import jax.numpy as jnp
from jax import lax
from jax.experimental import pallas as pl
from jax.experimental.pallas import tpu as pltpu

N_DEV = 32


def kernel(x, w_mat):
    m_glob, k_shard = x.shape
    k2, n = w_mat.shape
    assert k2 == k_shard
    m_chunk = m_glob // N_DEV
    n_half = n // 2

    def body(x_ref, w_ref, out_ref,
             comm_r, comm_l,
             send_sems_r, recv_sems_r,
             send_sems_l, recv_sems_l,
             credit_r, credit_l):
        my = lax.axis_index("i")
        right = lax.rem(my + 1, N_DEV)
        left = lax.rem(my + N_DEV - 1, N_DEV)

        barrier_sem = pltpu.get_barrier_semaphore()
        for nbr in (left, right):
            pl.semaphore_signal(barrier_sem, inc=1, device_id=(nbr,),
                                device_id_type=pl.DeviceIdType.MESH)
        pl.semaphore_wait(barrier_sem, 2)

        def partial(c, lo):
            rows = x_ref[pl.ds(c * m_chunk, m_chunk), :]
            return jnp.dot(rows, w_ref[:, lo:lo + n_half],
                           preferred_element_type=jnp.float32)

        dirs = (
            (comm_r, send_sems_r, recv_sems_r, credit_r, right, left,
             lambda t: lax.rem(my + N_DEV - 1 - t, N_DEV), 0),
            (comm_l, send_sems_l, recv_sems_l, credit_l, left, right,
             lambda t: lax.rem(my + 1 + t, N_DEV), n_half),
        )

        def make_rdma(comm, ssem, rsem, dst, t):
            return pltpu.make_async_remote_copy(
                src_ref=comm.at[t % 2],
                dst_ref=comm.at[(t + 1) % 2],
                send_sem=ssem.at[t % 2],
                recv_sem=rsem.at[(t + 1) % 2],
                device_id=(dst,),
                device_id_type=pl.DeviceIdType.MESH,
            )

        prev = []
        for comm, ssem, rsem, credit, dst, cdst, chunk_at, lo in dirs:
            comm[0] = partial(chunk_at(0), lo)
            rdma = make_rdma(comm, ssem, rsem, dst, 0)
            rdma.start()
            prev.append(rdma)

        for t in range(1, N_DEV - 1):
            s = t % 2
            ps = [partial(chunk_at(t), lo)
                  for _, _, _, _, _, _, chunk_at, lo in dirs]
            nxt = []
            for (comm, ssem, rsem, credit, dst, cdst, _, lo), p, rdma in zip(
                    dirs, ps, prev):
                rdma.wait()
                pl.semaphore_signal(credit, inc=1, device_id=(cdst,),
                                    device_id_type=pl.DeviceIdType.MESH)
                comm[s] = comm[s] + p
                pl.semaphore_wait(credit, 1)
                rdma2 = make_rdma(comm, ssem, rsem, dst, t)
                rdma2.start()
                nxt.append(rdma2)
            prev = nxt

        fs = (N_DEV - 1) % 2
        p_r = partial(my, 0)
        p_l = partial(my, n_half)
        for rdma in prev:
            rdma.wait()
        out_ref[:, 0:n_half] = comm_r[fs] + p_r
        out_ref[:, n_half:n] = comm_l[fs] + p_l

    return pl.pallas_call(
        body,
        out_shape=jax.ShapeDtypeStruct((m_chunk, n), jnp.float32),
        in_specs=[
            pl.BlockSpec(memory_space=pltpu.VMEM),
            pl.BlockSpec(memory_space=pltpu.VMEM),
        ],
        out_specs=pl.BlockSpec(memory_space=pltpu.VMEM),
        scratch_shapes=[
            pltpu.VMEM((2, m_chunk, n_half), jnp.float32),
            pltpu.VMEM((2, m_chunk, n_half), jnp.float32),
            pltpu.SemaphoreType.DMA((2,)),
            pltpu.SemaphoreType.DMA((2,)),
            pltpu.SemaphoreType.DMA((2,)),
            pltpu.SemaphoreType.DMA((2,)),
            pltpu.SemaphoreType.REGULAR,
            pltpu.SemaphoreType.REGULAR,
        ],
        compiler_params=pltpu.CompilerParams(collective_id=0),
    )(x, w_mat)


# device time: 785866 ns/iter; 1.9389x vs baseline; 1.8758x over previous
import jax
import jax.numpy as jnp
import numpy as np
from jax import lax
from jax.experimental import pallas as pl
from jax.experimental.pallas import tpu as pltpu

N_DEV = 32

_PLANE = [(0, 0), (1, 0), (1, 1), (0, 1), (0, 2), (1, 2), (1, 3), (0, 3)]


def _mesh_index(x, y, z):
    return 8 * z + _PLANE.index((x, y))


def _hamiltonian_perm():
    path_yz = []
    for z in range(4):
        ys = range(4) if z % 2 == 0 else range(3, -1, -1)
        path_yz.extend((y, z) for y in ys)
    ring = [(0, y, z) for (y, z) in path_yz]
    ring += [(1, y, z) for (y, z) in reversed(path_yz)]
    perm = [_mesh_index(*c) for c in ring]
    assert sorted(perm) == list(range(N_DEV))
    return perm


_PERM = _hamiltonian_perm()
_RINGPOS = [0] * N_DEV
for _k, _d in enumerate(_PERM):
    _RINGPOS[_d] = _k


def kernel(x, w_mat):
    m_glob, k_shard = x.shape
    k2, n = w_mat.shape
    assert k2 == k_shard
    m_chunk = m_glob // N_DEV
    n_half = n // 2

    def body(x_ref, w_ref, perm_ref, ringpos_ref, out_ref,
             comm_r, comm_l,
             send_sems_r, recv_sems_r,
             send_sems_l, recv_sems_l,
             credit_r, credit_l):
        my = lax.axis_index("i")
        q = ringpos_ref[my]
        nxt = perm_ref[lax.rem(q + 1, N_DEV)]
        prv = perm_ref[lax.rem(q + N_DEV - 1, N_DEV)]

        barrier_sem = pltpu.get_barrier_semaphore()
        for nbr in (prv, nxt):
            pl.semaphore_signal(barrier_sem, inc=1, device_id=(nbr,),
                                device_id_type=pl.DeviceIdType.MESH)
        pl.semaphore_wait(barrier_sem, 2)

        def partial(c, lo):
            rows = x_ref[pl.ds(c * m_chunk, m_chunk), :]
            return jnp.dot(rows, w_ref[:, lo:lo + n_half],
                           preferred_element_type=jnp.float32)

        def chunk_r(t):
            return perm_ref[lax.rem(q + 2 * N_DEV - 1 - t, N_DEV)]

        def chunk_l(t):
            return perm_ref[lax.rem(q + 1 + t, N_DEV)]

        dirs = (
            (comm_r, send_sems_r, recv_sems_r, credit_r, nxt, prv,
             chunk_r, 0),
            (comm_l, send_sems_l, recv_sems_l, credit_l, prv, nxt,
             chunk_l, n_half),
        )

        def make_rdma(comm, ssem, rsem, dst, t):
            return pltpu.make_async_remote_copy(
                src_ref=comm.at[t % 2],
                dst_ref=comm.at[(t + 1) % 2],
                send_sem=ssem.at[t % 2],
                recv_sem=rsem.at[(t + 1) % 2],
                device_id=(dst,),
                device_id_type=pl.DeviceIdType.MESH,
            )

        prev = []
        for comm, ssem, rsem, credit, dst, cdst, chunk_at, lo in dirs:
            comm[0] = partial(chunk_at(0), lo)
            rdma = make_rdma(comm, ssem, rsem, dst, 0)
            rdma.start()
            prev.append(rdma)

        for t in range(1, N_DEV - 1):
            s = t % 2
            ps = [partial(chunk_at(t), lo)
                  for _, _, _, _, _, _, chunk_at, lo in dirs]
            nxt_rdmas = []
            for (comm, ssem, rsem, credit, dst, cdst, _, lo), p, rdma in zip(
                    dirs, ps, prev):
                rdma.wait()
                pl.semaphore_signal(credit, inc=1, device_id=(cdst,),
                                    device_id_type=pl.DeviceIdType.MESH)
                comm[s] = comm[s] + p
                pl.semaphore_wait(credit, 1)
                rdma2 = make_rdma(comm, ssem, rsem, dst, t)
                rdma2.start()
                nxt_rdmas.append(rdma2)
            prev = nxt_rdmas

        fs = (N_DEV - 1) % 2
        p_r = partial(my, 0)
        p_l = partial(my, n_half)
        for rdma in prev:
            rdma.wait()
        out_ref[:, 0:n_half] = comm_r[fs] + p_r
        out_ref[:, n_half:n] = comm_l[fs] + p_l

    perm_arr = jnp.asarray(np.array(_PERM, dtype=np.int32))
    ringpos_arr = jnp.asarray(np.array(_RINGPOS, dtype=np.int32))

    return pl.pallas_call(
        body,
        out_shape=jax.ShapeDtypeStruct((m_chunk, n), jnp.float32),
        in_specs=[
            pl.BlockSpec(memory_space=pltpu.VMEM),
            pl.BlockSpec(memory_space=pltpu.VMEM),
            pl.BlockSpec(memory_space=pltpu.SMEM),
            pl.BlockSpec(memory_space=pltpu.SMEM),
        ],
        out_specs=pl.BlockSpec(memory_space=pltpu.VMEM),
        scratch_shapes=[
            pltpu.VMEM((2, m_chunk, n_half), jnp.float32),
            pltpu.VMEM((2, m_chunk, n_half), jnp.float32),
            pltpu.SemaphoreType.DMA((2,)),
            pltpu.SemaphoreType.DMA((2,)),
            pltpu.SemaphoreType.DMA((2,)),
            pltpu.SemaphoreType.DMA((2,)),
            pltpu.SemaphoreType.REGULAR,
            pltpu.SemaphoreType.REGULAR,
        ],
        compiler_params=pltpu.CompilerParams(collective_id=0),
    )(x, w_mat, perm_arr, ringpos_arr)


# device time: 712769 ns/iter; 2.1377x vs baseline; 1.1026x over previous
import jax
import jax.numpy as jnp
import numpy as np
from jax import lax
from jax.experimental import pallas as pl
from jax.experimental.pallas import tpu as pltpu

N_DEV = 32

_PLANE = [(0, 0), (1, 0), (1, 1), (0, 1), (0, 2), (1, 2), (1, 3), (0, 3)]


def _mesh_index(x, y, z):
    return 8 * z + _PLANE.index((x, y))


def _hamiltonian_perm():
    path_yz = []
    for z in range(4):
        ys = range(4) if z % 2 == 0 else range(3, -1, -1)
        path_yz.extend((y, z) for y in ys)
    ring = [(0, y, z) for (y, z) in path_yz]
    ring += [(1, y, z) for (y, z) in reversed(path_yz)]
    perm = [_mesh_index(*c) for c in ring]
    assert sorted(perm) == list(range(N_DEV))
    return perm


_PERM = _hamiltonian_perm()
_RINGPOS = [0] * N_DEV
for _k, _d in enumerate(_PERM):
    _RINGPOS[_d] = _k


def kernel(x, w_mat):
    m_glob, k_shard = x.shape
    k2, n = w_mat.shape
    assert k2 == k_shard
    m_chunk = m_glob // N_DEV
    n_half = n // 2
    n_quart = n // 4

    def body(x_ref, w_ref, perm_ref, ringpos_ref, out_ref,
             comm_ra, comm_la, comm_rb, comm_lb,
             ssem_ra, rsem_ra, ssem_la, rsem_la,
             ssem_rb, rsem_rb, ssem_lb, rsem_lb,
             credit_ra, credit_la, credit_rb, credit_lb):
        my = lax.axis_index("i")
        q = ringpos_ref[my]
        nxt = perm_ref[lax.rem(q + 1, N_DEV)]
        prv = perm_ref[lax.rem(q + N_DEV - 1, N_DEV)]

        barrier_sem = pltpu.get_barrier_semaphore()
        for nbr in (prv, nxt):
            pl.semaphore_signal(barrier_sem, inc=1, device_id=(nbr,),
                                device_id_type=pl.DeviceIdType.MESH)
        pl.semaphore_wait(barrier_sem, 2)

        def partial(c, lo):
            rows = x_ref[pl.ds(c * m_chunk, m_chunk), :]
            return jnp.dot(rows, w_ref[:, lo:lo + n_half],
                           preferred_element_type=jnp.float32)

        def chunk_r(t):
            return perm_ref[lax.rem(q + 2 * N_DEV - 1 - t, N_DEV)]

        def chunk_l(t):
            return perm_ref[lax.rem(q + 1 + t, N_DEV)]

        lanes = (
            (comm_ra, ssem_ra, rsem_ra, credit_ra, nxt, prv, 0, 0),
            (comm_la, ssem_la, rsem_la, credit_la, prv, nxt, 1, 0),
            (comm_rb, ssem_rb, rsem_rb, credit_rb, nxt, prv, 0, n_quart),
            (comm_lb, ssem_lb, rsem_lb, credit_lb, prv, nxt, 1, n_quart),
        )
        chunk_fns = (chunk_r, chunk_l)
        col_base = (0, n_half)

        def make_rdma(comm, ssem, rsem, dst, t):
            return pltpu.make_async_remote_copy(
                src_ref=comm.at[t % 2],
                dst_ref=comm.at[(t + 1) % 2],
                send_sem=ssem.at[t % 2],
                recv_sem=rsem.at[(t + 1) % 2],
                device_id=(dst,),
                device_id_type=pl.DeviceIdType.MESH,
            )

        ps = [partial(chunk_fns[d](0), lob) for d, lob in
              zip((0, 1), col_base)]
        prev = []
        for comm, ssem, rsem, credit, dst, cdst, d, lo in lanes:
            comm[0] = ps[d][:, lo:lo + n_quart]
            rdma = make_rdma(comm, ssem, rsem, dst, 0)
            rdma.start()
            prev.append(rdma)

        for t in range(1, N_DEV - 1):
            s = t % 2
            ps = [partial(chunk_fns[d](t), lob) for d, lob in
                  zip((0, 1), col_base)]
            nxt_rdmas = []
            for (comm, ssem, rsem, credit, dst, cdst, d, lo), rdma in zip(
                    lanes, prev):
                rdma.wait()
                pl.semaphore_signal(credit, inc=1, device_id=(cdst,),
                                    device_id_type=pl.DeviceIdType.MESH)
                comm[s] = comm[s] + ps[d][:, lo:lo + n_quart]
                pl.semaphore_wait(credit, 1)
                rdma2 = make_rdma(comm, ssem, rsem, dst, t)
                rdma2.start()
                nxt_rdmas.append(rdma2)
            prev = nxt_rdmas

        fs = (N_DEV - 1) % 2
        ps = [partial(my, lob) for lob in col_base]
        for (comm, ssem, rsem, credit, dst, cdst, d, lo), rdma in zip(
                lanes, prev):
            rdma.wait()
            out_ref[:, col_base[d] + lo:col_base[d] + lo + n_quart] = (
                comm[fs] + ps[d][:, lo:lo + n_quart])

    perm_arr = jnp.asarray(np.array(_PERM, dtype=np.int32))
    ringpos_arr = jnp.asarray(np.array(_RINGPOS, dtype=np.int32))

    return pl.pallas_call(
        body,
        out_shape=jax.ShapeDtypeStruct((m_chunk, n), jnp.float32),
        in_specs=[
            pl.BlockSpec(memory_space=pltpu.VMEM),
            pl.BlockSpec(memory_space=pltpu.VMEM),
            pl.BlockSpec(memory_space=pltpu.SMEM),
            pl.BlockSpec(memory_space=pltpu.SMEM),
        ],
        out_specs=pl.BlockSpec(memory_space=pltpu.VMEM),
        scratch_shapes=(
            [pltpu.VMEM((2, m_chunk, n_quart), jnp.float32)] * 4
            + [pltpu.SemaphoreType.DMA((2,))] * 8
            + [pltpu.SemaphoreType.REGULAR] * 4
        ),
        compiler_params=pltpu.CompilerParams(collective_id=0),
    )(x, w_mat, perm_arr, ringpos_arr)
